# baseline (speedup 1.0000x reference)
import functools

import jax
import jax.numpy as jnp
from jax import lax
from jax.experimental import pallas as pl
from jax.experimental.pallas import tpu as pltpu
from jax.experimental.pallas import tpu_sc as plsc

_N = 10000
_E = 320000
_H = 128
_G = 64

_LANES = 128
_TILES = 32
_EPAD = 327680
_ROWS = _EPAD // _LANES
_RPT = _ROWS // _TILES
_PHASES = 2
_RPP = _RPT // _PHASES
_JUNK = 112
_NACC = _N + _JUNK
_ZROWS = _NACC // 16
_NDEG = 10240
_DPT = _NDEG // 16
_BN = 2000
_GRID = _N // _BN


def _nr_rsqrt(y):
  r = lax.rsqrt(y)
  r = 0.5 * r * (3.0 - y * r * r)
  return 0.5 * r * (3.0 - y * r * r)


def _nr_recip(y):
  r = _nr_rsqrt(y)
  return r * r

def _sc_mesh():
  return plsc.VectorSubcoreMesh(core_axis_name="c", subcore_axis_name="s")


def _deg_partials(dst2d, zeros1d):
  @functools.partial(
      pl.kernel,
      out_type=jax.ShapeDtypeStruct((2 * _NDEG,), jnp.float32),
      mesh=_sc_mesh(),
      scratch_types=[
          pltpu.VMEM((_RPT, _LANES), jnp.int32),
          pltpu.VMEM((_LANES,), jnp.float32),
          pltpu.VMEM_SHARED((_NDEG,), jnp.float32),
      ],
  )
  def k(dst_hbm, z_hbm, out_hbm, idx_v, ones_v, deg_sh):
    c = lax.axis_index("c")
    s = lax.axis_index("s")
    t = c * 16 + s
    pltpu.sync_copy(z_hbm, deg_sh.at[pl.ds(s * _DPT, _DPT)])
    for j in range(_LANES // 16):
      ones_v[pl.ds(j * 16, 16)] = jnp.full((16,), 1.0, jnp.float32)
    pltpu.sync_copy(dst_hbm.at[pl.ds(t * _RPT, _RPT)], idx_v)
    plsc.subcore_barrier()

    def body(g, carry):
      pltpu.sync_copy(ones_v, deg_sh.at[idx_v.at[g]], add=True)
      return carry

    lax.fori_loop(0, _RPT, body, 0)
    plsc.subcore_barrier()
    pltpu.sync_copy(
        deg_sh.at[pl.ds(s * _DPT, _DPT)],
        out_hbm.at[pl.ds(c * _NDEG + s * _DPT, _DPT)],
    )

  return k(dst2d, zeros1d)


def _msg_partials(h, src2d, dst2d, zeros2d):
  @functools.partial(
      pl.kernel,
      out_type=jax.ShapeDtypeStruct((2, _NACC, _H), jnp.float32),
      mesh=_sc_mesh(),
      scratch_types=[
          pltpu.VMEM((_RPP, _LANES), jnp.int32),
          pltpu.VMEM((_RPP, _LANES), jnp.int32),
          pltpu.VMEM((2, _LANES, _H), jnp.float32),
          pltpu.VMEM_SHARED((_NACC, _H), jnp.float32),
          pltpu.SemaphoreType.DMA,
      ],
  )
  def k(h_hbm, src_hbm, dst_hbm, z_hbm, out_hbm, src_v, dst_v, rows_v, acc_sh,
        sem):
    c = lax.axis_index("c")
    s = lax.axis_index("s")
    t = c * 16 + s
    pltpu.sync_copy(
        z_hbm.at[pl.ds(0, _ZROWS)], acc_sh.at[pl.ds(s * _ZROWS, _ZROWS)]
    )
    plsc.subcore_barrier()

    r0 = rows_v.at[0]
    r1 = rows_v.at[1]

    def gather(g, r):
      return pltpu.async_copy(h_hbm.at[src_v.at[g]], r, sem)

    def gwait(g, r):
      pltpu.make_async_copy(h_hbm.at[src_v.at[g]], r, sem).wait()

    def scatter(g, r):
      pltpu.sync_copy(r, acc_sh.at[dst_v.at[g]], add=True)

    for ph in range(_PHASES):
      base = t * _RPT + ph * _RPP
      pltpu.sync_copy(src_hbm.at[pl.ds(base, _RPP)], src_v)
      pltpu.sync_copy(dst_hbm.at[pl.ds(base, _RPP)], dst_v)
      gather(0, r0)

      def body(j, carry):
        g = 2 * j
        gwait(g, r0)
        gather(g + 1, r1)
        scatter(g, r0)
        gwait(g + 1, r1)
        gather(g + 2, r0)
        scatter(g + 1, r1)
        return carry

      lax.fori_loop(0, _RPP // 2 - 1, body, 0)
      gL = _RPP - 2
      gwait(gL, r0)
      gather(gL + 1, r1)
      scatter(gL, r0)
      gwait(gL + 1, r1)
      scatter(gL + 1, r1)
    plsc.subcore_barrier()
    pltpu.sync_copy(
        acc_sh.at[pl.ds(s * _ZROWS, _ZROWS)],
        out_hbm.at[c, pl.ds(s * _ZROWS, _ZROWS)],
    )

  return k(h, src2d, dst2d, zeros2d)


def _tc_first(degp0, degp1, x, W1):
  def body(d0_ref, d1_ref, x_ref, w_ref, dinv_ref, h_ref):
    dinv = _nr_rsqrt(d0_ref[...] + d1_ref[...] + 1.0)
    dinv_ref[...] = dinv
    h_ref[...] = dinv * jnp.dot(
        x_ref[...], w_ref[...], preferred_element_type=jnp.float32
    )

  return pl.pallas_call(
      body,
      grid=(_GRID,),
      in_specs=[
          pl.BlockSpec((_BN, 1), lambda i: (i, 0)),
          pl.BlockSpec((_BN, 1), lambda i: (i, 0)),
          pl.BlockSpec((_BN, _H), lambda i: (i, 0)),
          pl.BlockSpec((_H, _H), lambda i: (0, 0)),
      ],
      out_specs=[
          pl.BlockSpec((_BN, 1), lambda i: (i, 0)),
          pl.BlockSpec((_BN, _H), lambda i: (i, 0)),
      ],
      out_shape=[
          jax.ShapeDtypeStruct((_N, 1), jnp.float32),
          jax.ShapeDtypeStruct((_N, _H), jnp.float32),
      ],
  )(degp0, degp1, x, W1)


def _tc_layer(p, h, dinv, b2d, Wn):
  def body(p_ref, h_ref, dinv_ref, b_ref, w_ref, o_ref):
    acc = p_ref[0] + p_ref[1] + h_ref[...]
    xl = jnp.maximum(dinv_ref[...] * acc + b_ref[...], 0.0)
    o_ref[...] = dinv_ref[...] * jnp.dot(
        xl, w_ref[...], preferred_element_type=jnp.float32
    )

  return pl.pallas_call(
      body,
      grid=(_GRID,),
      in_specs=[
          pl.BlockSpec((2, _BN, _H), lambda i: (0, i, 0)),
          pl.BlockSpec((_BN, _H), lambda i: (i, 0)),
          pl.BlockSpec((_BN, 1), lambda i: (i, 0)),
          pl.BlockSpec((1, _H), lambda i: (0, 0)),
          pl.BlockSpec((_H, _H), lambda i: (0, 0)),
      ],
      out_specs=pl.BlockSpec((_BN, _H), lambda i: (i, 0)),
      out_shape=jax.ShapeDtypeStruct((_N, _H), jnp.float32),
  )(p, h, dinv, b2d, Wn)


def _tc_final(p, h, dinv, b2d, batch3, gm, gb, mu, var, w1, b1, w2, b2):
  def body(p_ref, h_ref, dinv_ref, b_ref, batch_ref, gm_ref, gb_ref, mu_ref,
           var_ref, w1_ref, b1_ref, w2_ref, b2_ref, xb_ref, out_ref, sums,
           counts):
    i = pl.program_id(0)

    @pl.when(i == 0)
    def _():
      sums[...] = jnp.zeros_like(sums)
      counts[...] = jnp.zeros_like(counts)

    h3 = jnp.maximum(
        dinv_ref[...] * (p_ref[0] + p_ref[1] + h_ref[...]) + b_ref[...], 0.0
    )
    g = batch_ref[0, 0, :]
    onehot = (
        g[None, :] == lax.broadcasted_iota(jnp.int32, (_G, _BN), 0)
    ).astype(jnp.float32)
    sums[...] += jnp.dot(onehot, h3, preferred_element_type=jnp.float32, precision=lax.Precision.HIGHEST)
    counts[...] += jnp.sum(onehot, axis=1, keepdims=True)

    @pl.when(i == _GRID - 1)
    def _():
      pooled = sums[...] * _nr_recip(jnp.maximum(counts[...], 1.0))
      xb = (pooled - mu_ref[...]) * _nr_rsqrt(var_ref[...] + 1e-5) * gm_ref[
          ...
      ] + gb_ref[...]
      xb_ref[...] = xb
      t1 = jnp.maximum(
          jnp.dot(xb, w1_ref[...], preferred_element_type=jnp.float32)
          + b1_ref[...],
          0.0,
      )
      out_ref[...] = jnp.maximum(
          jnp.dot(t1, w2_ref[...], preferred_element_type=jnp.float32)
          + b2_ref[...],
          0.0,
      )

  return pl.pallas_call(
      body,
      grid=(_GRID,),
      in_specs=[
          pl.BlockSpec((2, _BN, _H), lambda i: (0, i, 0)),
          pl.BlockSpec((_BN, _H), lambda i: (i, 0)),
          pl.BlockSpec((_BN, 1), lambda i: (i, 0)),
          pl.BlockSpec((1, _H), lambda i: (0, 0)),
          pl.BlockSpec((1, 1, _BN), lambda i: (i, 0, 0)),
          pl.BlockSpec((1, _H), lambda i: (0, 0)),
          pl.BlockSpec((1, _H), lambda i: (0, 0)),
          pl.BlockSpec((1, _H), lambda i: (0, 0)),
          pl.BlockSpec((1, _H), lambda i: (0, 0)),
          pl.BlockSpec((_H, _H // 2), lambda i: (0, 0)),
          pl.BlockSpec((1, _H // 2), lambda i: (0, 0)),
          pl.BlockSpec((_H // 2, 2), lambda i: (0, 0)),
          pl.BlockSpec((1, 2), lambda i: (0, 0)),
      ],
      out_specs=[
          pl.BlockSpec((_G, _H), lambda i: (0, 0)),
          pl.BlockSpec((_G, 2), lambda i: (0, 0)),
      ],
      out_shape=[
          jax.ShapeDtypeStruct((_G, _H), jnp.float32),
          jax.ShapeDtypeStruct((_G, 2), jnp.float32),
      ],
      scratch_shapes=[
          pltpu.VMEM((_G, _H), jnp.float32),
          pltpu.VMEM((_G, 1), jnp.float32),
      ],
  )(p, h, dinv, b2d, batch3, gm, gb, mu, var, w1, b1, w2, b2)


def kernel(x, edge_index, _, batch, W1, b1, W2, b2, W3, b3, bn_gamma, bn_beta,
           bn_mean, bn_var, lin1_W, lin1_b, lin2_W, lin2_b):
  src = edge_index[0]
  dst = edge_index[1]
  pad = _EPAD - _E
  ar = jnp.arange(pad, dtype=jnp.int32)
  pad_src = (ar * 613) % _N
  pad_dst = _N + (ar % _JUNK)
  src2d = jnp.concatenate([src, pad_src]).reshape(_ROWS, _LANES)
  dst2d = jnp.concatenate([dst, pad_dst]).reshape(_ROWS, _LANES)
  zeros1d = jnp.zeros((_DPT,), jnp.float32)
  zeros2d = jnp.zeros((_ZROWS, _H), jnp.float32)

  degp = _deg_partials(dst2d, zeros1d)
  dinv, h1 = _tc_first(
      degp[:_N, None], degp[_NDEG : _NDEG + _N, None], x, W1
  )
  p1 = _msg_partials(h1, src2d, dst2d, zeros2d)
  h2 = _tc_layer(p1, h1, dinv, b1[None, :], W2)
  p2 = _msg_partials(h2, src2d, dst2d, zeros2d)
  h3 = _tc_layer(p2, h2, dinv, b2[None, :], W3)
  p3 = _msg_partials(h3, src2d, dst2d, zeros2d)
  xb, out = _tc_final(
      p3, h3, dinv, b3[None, :],
      batch.reshape(_GRID, 1, _BN),
      bn_gamma[None, :], bn_beta[None, :], bn_mean[None, :], bn_var[None, :],
      lin1_W, lin1_b[None, :], lin2_W, lin2_b[None, :],
  )
  return (xb, out)


# speedup vs baseline: 1.1452x; 1.1452x over previous
import functools

import jax
import jax.numpy as jnp
from jax import lax
from jax.experimental import pallas as pl
from jax.experimental.pallas import tpu as pltpu
from jax.experimental.pallas import tpu_sc as plsc

_N = 10000
_E = 320000
_H = 128
_G = 64

_LANES = 128
_TILES = 32
_EPAD = 327680
_ROWS = _EPAD // _LANES
_RPT = _ROWS // _TILES
_PHASES = 2
_RPP = _RPT // _PHASES
_JUNK = 112
_NACC = _N + _JUNK
_ZROWS = _NACC // 16
_NDEG = 10240
_DPT = _NDEG // 16
_BN = 2000
_GRID = _N // _BN


def _nr_rsqrt(y):
  r = lax.rsqrt(y)
  r = 0.5 * r * (3.0 - y * r * r)
  return 0.5 * r * (3.0 - y * r * r)


def _nr_recip(y):
  r = _nr_rsqrt(y)
  return r * r

def _sc_mesh():
  return plsc.VectorSubcoreMesh(core_axis_name="c", subcore_axis_name="s")


def _deg_partials(dst2d, zeros1d):
  @functools.partial(
      pl.kernel,
      out_type=jax.ShapeDtypeStruct((2 * _NDEG,), jnp.float32),
      mesh=_sc_mesh(),
      scratch_types=[
          pltpu.VMEM((_RPT, _LANES), jnp.int32),
          pltpu.VMEM((_LANES,), jnp.float32),
          pltpu.VMEM_SHARED((_NDEG,), jnp.float32),
      ],
  )
  def k(dst_hbm, z_hbm, out_hbm, idx_v, ones_v, deg_sh):
    c = lax.axis_index("c")
    s = lax.axis_index("s")
    t = c * 16 + s
    pltpu.sync_copy(z_hbm, deg_sh.at[pl.ds(s * _DPT, _DPT)])
    for j in range(_LANES // 16):
      ones_v[pl.ds(j * 16, 16)] = jnp.full((16,), 1.0, jnp.float32)
    pltpu.sync_copy(dst_hbm.at[pl.ds(t * _RPT, _RPT)], idx_v)
    plsc.subcore_barrier()

    def body(g, carry):
      pltpu.sync_copy(ones_v, deg_sh.at[idx_v.at[g]], add=True)
      return carry

    lax.fori_loop(0, _RPT, body, 0)
    plsc.subcore_barrier()
    pltpu.sync_copy(
        deg_sh.at[pl.ds(s * _DPT, _DPT)],
        out_hbm.at[pl.ds(c * _NDEG + s * _DPT, _DPT)],
    )

  return k(dst2d, zeros1d)


def _msg_partials(h, src2d, dst2d, zeros2d):
  @functools.partial(
      pl.kernel,
      out_type=jax.ShapeDtypeStruct((2, _NACC, _H), jnp.float32),
      mesh=_sc_mesh(),
      scratch_types=[
          pltpu.VMEM((_RPP, _LANES), jnp.int32),
          pltpu.VMEM((_RPP, _LANES), jnp.int32),
          pltpu.VMEM((2, _LANES, _H), jnp.float32),
          pltpu.VMEM_SHARED((_NACC, _H), jnp.float32),
          pltpu.SemaphoreType.DMA,
          pltpu.SemaphoreType.DMA,
      ],
  )
  def k(h_hbm, src_hbm, dst_hbm, z_hbm, out_hbm, src_v, dst_v, rows_v, acc_sh,
        sem0, sem1):
    c = lax.axis_index("c")
    s = lax.axis_index("s")
    t = c * 16 + s
    pltpu.sync_copy(
        z_hbm.at[pl.ds(0, _ZROWS)], acc_sh.at[pl.ds(s * _ZROWS, _ZROWS)]
    )
    plsc.subcore_barrier()

    r0 = rows_v.at[0]
    r1 = rows_v.at[1]

    def gather(g, r, sem):
      return pltpu.async_copy(h_hbm.at[src_v.at[g]], r, sem)

    def gwait(g, r, sem):
      pltpu.make_async_copy(h_hbm.at[src_v.at[g]], r, sem).wait()

    def scatter(g, r):
      pltpu.sync_copy(r, acc_sh.at[dst_v.at[g]], add=True)

    for ph in range(_PHASES):
      base = t * _RPT + ph * _RPP
      pltpu.sync_copy(src_hbm.at[pl.ds(base, _RPP)], src_v)
      pltpu.sync_copy(dst_hbm.at[pl.ds(base, _RPP)], dst_v)
      gather(0, r0, sem0)
      gather(1, r1, sem1)

      def body(j, carry):
        g = 2 * j
        gwait(g, r0, sem0)
        scatter(g, r0)
        gather(g + 2, r0, sem0)
        gwait(g + 1, r1, sem1)
        scatter(g + 1, r1)
        gather(g + 3, r1, sem1)
        return carry

      lax.fori_loop(0, _RPP // 2 - 1, body, 0)
      gL = _RPP - 2
      gwait(gL, r0, sem0)
      scatter(gL, r0)
      gwait(gL + 1, r1, sem1)
      scatter(gL + 1, r1)
    plsc.subcore_barrier()
    pltpu.sync_copy(
        acc_sh.at[pl.ds(s * _ZROWS, _ZROWS)],
        out_hbm.at[c, pl.ds(s * _ZROWS, _ZROWS)],
    )

  return k(h, src2d, dst2d, zeros2d)


def _tc_first(degp0, degp1, x, W1):
  def body(d0_ref, d1_ref, x_ref, w_ref, dinv_ref, h_ref):
    dinv = _nr_rsqrt(d0_ref[...] + d1_ref[...] + 1.0)
    dinv_ref[...] = dinv
    h_ref[...] = dinv * jnp.dot(
        x_ref[...], w_ref[...], preferred_element_type=jnp.float32
    )

  return pl.pallas_call(
      body,
      grid=(_GRID,),
      in_specs=[
          pl.BlockSpec((_BN, 1), lambda i: (i, 0)),
          pl.BlockSpec((_BN, 1), lambda i: (i, 0)),
          pl.BlockSpec((_BN, _H), lambda i: (i, 0)),
          pl.BlockSpec((_H, _H), lambda i: (0, 0)),
      ],
      out_specs=[
          pl.BlockSpec((_BN, 1), lambda i: (i, 0)),
          pl.BlockSpec((_BN, _H), lambda i: (i, 0)),
      ],
      out_shape=[
          jax.ShapeDtypeStruct((_N, 1), jnp.float32),
          jax.ShapeDtypeStruct((_N, _H), jnp.float32),
      ],
  )(degp0, degp1, x, W1)


def _tc_layer(p, h, dinv, b2d, Wn):
  def body(p_ref, h_ref, dinv_ref, b_ref, w_ref, o_ref):
    acc = p_ref[0] + p_ref[1] + h_ref[...]
    xl = jnp.maximum(dinv_ref[...] * acc + b_ref[...], 0.0)
    o_ref[...] = dinv_ref[...] * jnp.dot(
        xl, w_ref[...], preferred_element_type=jnp.float32
    )

  return pl.pallas_call(
      body,
      grid=(_GRID,),
      in_specs=[
          pl.BlockSpec((2, _BN, _H), lambda i: (0, i, 0)),
          pl.BlockSpec((_BN, _H), lambda i: (i, 0)),
          pl.BlockSpec((_BN, 1), lambda i: (i, 0)),
          pl.BlockSpec((1, _H), lambda i: (0, 0)),
          pl.BlockSpec((_H, _H), lambda i: (0, 0)),
      ],
      out_specs=pl.BlockSpec((_BN, _H), lambda i: (i, 0)),
      out_shape=jax.ShapeDtypeStruct((_N, _H), jnp.float32),
  )(p, h, dinv, b2d, Wn)


def _tc_final(p, h, dinv, b2d, batch3, gm, gb, mu, var, w1, b1, w2, b2):
  def body(p_ref, h_ref, dinv_ref, b_ref, batch_ref, gm_ref, gb_ref, mu_ref,
           var_ref, w1_ref, b1_ref, w2_ref, b2_ref, xb_ref, out_ref, sums,
           counts):
    i = pl.program_id(0)

    @pl.when(i == 0)
    def _():
      sums[...] = jnp.zeros_like(sums)
      counts[...] = jnp.zeros_like(counts)

    h3 = jnp.maximum(
        dinv_ref[...] * (p_ref[0] + p_ref[1] + h_ref[...]) + b_ref[...], 0.0
    )
    g = batch_ref[0, 0, :]
    onehot = (
        g[None, :] == lax.broadcasted_iota(jnp.int32, (_G, _BN), 0)
    ).astype(jnp.float32)
    sums[...] += jnp.dot(onehot, h3, preferred_element_type=jnp.float32, precision=lax.Precision.HIGHEST)
    counts[...] += jnp.sum(onehot, axis=1, keepdims=True)

    @pl.when(i == _GRID - 1)
    def _():
      pooled = sums[...] * _nr_recip(jnp.maximum(counts[...], 1.0))
      xb = (pooled - mu_ref[...]) * _nr_rsqrt(var_ref[...] + 1e-5) * gm_ref[
          ...
      ] + gb_ref[...]
      xb_ref[...] = xb
      t1 = jnp.maximum(
          jnp.dot(xb, w1_ref[...], preferred_element_type=jnp.float32)
          + b1_ref[...],
          0.0,
      )
      out_ref[...] = jnp.maximum(
          jnp.dot(t1, w2_ref[...], preferred_element_type=jnp.float32)
          + b2_ref[...],
          0.0,
      )

  return pl.pallas_call(
      body,
      grid=(_GRID,),
      in_specs=[
          pl.BlockSpec((2, _BN, _H), lambda i: (0, i, 0)),
          pl.BlockSpec((_BN, _H), lambda i: (i, 0)),
          pl.BlockSpec((_BN, 1), lambda i: (i, 0)),
          pl.BlockSpec((1, _H), lambda i: (0, 0)),
          pl.BlockSpec((1, 1, _BN), lambda i: (i, 0, 0)),
          pl.BlockSpec((1, _H), lambda i: (0, 0)),
          pl.BlockSpec((1, _H), lambda i: (0, 0)),
          pl.BlockSpec((1, _H), lambda i: (0, 0)),
          pl.BlockSpec((1, _H), lambda i: (0, 0)),
          pl.BlockSpec((_H, _H // 2), lambda i: (0, 0)),
          pl.BlockSpec((1, _H // 2), lambda i: (0, 0)),
          pl.BlockSpec((_H // 2, 2), lambda i: (0, 0)),
          pl.BlockSpec((1, 2), lambda i: (0, 0)),
      ],
      out_specs=[
          pl.BlockSpec((_G, _H), lambda i: (0, 0)),
          pl.BlockSpec((_G, 2), lambda i: (0, 0)),
      ],
      out_shape=[
          jax.ShapeDtypeStruct((_G, _H), jnp.float32),
          jax.ShapeDtypeStruct((_G, 2), jnp.float32),
      ],
      scratch_shapes=[
          pltpu.VMEM((_G, _H), jnp.float32),
          pltpu.VMEM((_G, 1), jnp.float32),
      ],
  )(p, h, dinv, b2d, batch3, gm, gb, mu, var, w1, b1, w2, b2)


def kernel(x, edge_index, _, batch, W1, b1, W2, b2, W3, b3, bn_gamma, bn_beta,
           bn_mean, bn_var, lin1_W, lin1_b, lin2_W, lin2_b):
  src = edge_index[0]
  dst = edge_index[1]
  pad = _EPAD - _E
  ar = jnp.arange(pad, dtype=jnp.int32)
  pad_src = (ar * 613) % _N
  pad_dst = _N + (ar % _JUNK)
  src2d = jnp.concatenate([src, pad_src]).reshape(_ROWS, _LANES)
  dst2d = jnp.concatenate([dst, pad_dst]).reshape(_ROWS, _LANES)
  zeros1d = jnp.zeros((_DPT,), jnp.float32)
  zeros2d = jnp.zeros((_ZROWS, _H), jnp.float32)

  degp = _deg_partials(dst2d, zeros1d)
  dinv, h1 = _tc_first(
      degp[:_N, None], degp[_NDEG : _NDEG + _N, None], x, W1
  )
  p1 = _msg_partials(h1, src2d, dst2d, zeros2d)
  h2 = _tc_layer(p1, h1, dinv, b1[None, :], W2)
  p2 = _msg_partials(h2, src2d, dst2d, zeros2d)
  h3 = _tc_layer(p2, h2, dinv, b2[None, :], W3)
  p3 = _msg_partials(h3, src2d, dst2d, zeros2d)
  xb, out = _tc_final(
      p3, h3, dinv, b3[None, :],
      batch.reshape(_GRID, 1, _BN),
      bn_gamma[None, :], bn_beta[None, :], bn_mean[None, :], bn_var[None, :],
      lin1_W, lin1_b[None, :], lin2_W, lin2_b[None, :],
  )
  return (xb, out)


# speedup vs baseline: 1.1847x; 1.0345x over previous
import functools

import jax
import jax.numpy as jnp
from jax import lax
from jax.experimental import pallas as pl
from jax.experimental.pallas import tpu as pltpu
from jax.experimental.pallas import tpu_sc as plsc

_N = 10000
_E = 320000
_H = 128
_G = 64

_LANES = 128
_TILES = 32
_EPAD = 327680
_ROWS = _EPAD // _LANES
_RPT = _ROWS // _TILES
_PHASES = 2
_RPP = _RPT // _PHASES
_JUNK = 112
_NACC = _N + _JUNK
_ZROWS = _NACC // 16
_NDEG = 10240
_DPT = _NDEG // 16
_BN = 2000
_GRID = _N // _BN


def _nr_rsqrt(y):
  r = lax.rsqrt(y)
  r = 0.5 * r * (3.0 - y * r * r)
  return 0.5 * r * (3.0 - y * r * r)


def _nr_recip(y):
  r = _nr_rsqrt(y)
  return r * r

def _sc_mesh():
  return plsc.VectorSubcoreMesh(core_axis_name="c", subcore_axis_name="s")


def _deg_partials(dst2d, zeros1d):
  @functools.partial(
      pl.kernel,
      out_type=jax.ShapeDtypeStruct((2 * _NDEG,), jnp.float32),
      mesh=_sc_mesh(),
      scratch_types=[
          pltpu.VMEM((_RPT, _LANES), jnp.int32),
          pltpu.VMEM((_LANES,), jnp.float32),
          pltpu.VMEM_SHARED((_NDEG,), jnp.float32),
          pltpu.SemaphoreType.DMA,
      ],
  )
  def k(dst_hbm, z_hbm, out_hbm, idx_v, ones_v, deg_sh, sem):
    c = lax.axis_index("c")
    s = lax.axis_index("s")
    t = c * 16 + s
    pltpu.sync_copy(z_hbm, deg_sh.at[pl.ds(s * _DPT, _DPT)])
    for j in range(_LANES // 16):
      ones_v[pl.ds(j * 16, 16)] = jnp.full((16,), 1.0, jnp.float32)
    pltpu.sync_copy(dst_hbm.at[pl.ds(t * _RPT, _RPT)], idx_v)
    plsc.subcore_barrier()

    def fire(g, carry):
      pltpu.async_copy(ones_v, deg_sh.at[idx_v.at[g]], sem, add=True)
      return carry

    lax.fori_loop(0, _RPT, fire, 0)

    def drain(g, carry):
      pltpu.make_async_copy(ones_v, deg_sh.at[idx_v.at[g]], sem).wait()
      return carry

    lax.fori_loop(0, _RPT, drain, 0)
    plsc.subcore_barrier()
    pltpu.sync_copy(
        deg_sh.at[pl.ds(s * _DPT, _DPT)],
        out_hbm.at[pl.ds(c * _NDEG + s * _DPT, _DPT)],
    )

  return k(dst2d, zeros1d)


def _msg_partials(h, src2d, dst2d, zeros2d):
  @functools.partial(
      pl.kernel,
      out_type=jax.ShapeDtypeStruct((2, _NACC, _H), jnp.float32),
      mesh=_sc_mesh(),
      scratch_types=[
          pltpu.VMEM((_RPP, _LANES), jnp.int32),
          pltpu.VMEM((_RPP, _LANES), jnp.int32),
          pltpu.VMEM((2, _LANES, _H), jnp.float32),
          pltpu.VMEM_SHARED((_NACC, _H), jnp.float32),
          pltpu.SemaphoreType.DMA,
          pltpu.SemaphoreType.DMA,
          pltpu.SemaphoreType.DMA,
      ],
  )
  def k(h_hbm, src_hbm, dst_hbm, z_hbm, out_hbm, src_v, dst_v, rows_v, acc_sh,
        sem0, sem1, semz):
    c = lax.axis_index("c")
    s = lax.axis_index("s")
    t = c * 16 + s
    zdesc = pltpu.async_copy(
        z_hbm.at[pl.ds(0, _ZROWS)], acc_sh.at[pl.ds(s * _ZROWS, _ZROWS)], semz
    )

    r0 = rows_v.at[0]
    r1 = rows_v.at[1]

    def gather(g, r, sem):
      return pltpu.async_copy(h_hbm.at[src_v.at[g]], r, sem)

    def gwait(g, r, sem):
      pltpu.make_async_copy(h_hbm.at[src_v.at[g]], r, sem).wait()

    def scatter(g, r):
      pltpu.sync_copy(r, acc_sh.at[dst_v.at[g]], add=True)

    for ph in range(_PHASES):
      base = t * _RPT + ph * _RPP
      pltpu.sync_copy(src_hbm.at[pl.ds(base, _RPP)], src_v)
      pltpu.sync_copy(dst_hbm.at[pl.ds(base, _RPP)], dst_v)
      gather(0, r0, sem0)
      gather(1, r1, sem1)
      if ph == 0:
        zdesc.wait()
        plsc.subcore_barrier()

      def body(j, carry):
        g = 2 * j
        gwait(g, r0, sem0)
        scatter(g, r0)
        gather(g + 2, r0, sem0)
        gwait(g + 1, r1, sem1)
        scatter(g + 1, r1)
        gather(g + 3, r1, sem1)
        return carry

      lax.fori_loop(0, _RPP // 2 - 1, body, 0)
      gL = _RPP - 2
      gwait(gL, r0, sem0)
      scatter(gL, r0)
      gwait(gL + 1, r1, sem1)
      scatter(gL + 1, r1)
    plsc.subcore_barrier()
    pltpu.sync_copy(
        acc_sh.at[pl.ds(s * _ZROWS, _ZROWS)],
        out_hbm.at[c, pl.ds(s * _ZROWS, _ZROWS)],
    )

  return k(h, src2d, dst2d, zeros2d)


def _tc_first(degp0, degp1, x, W1):
  def body(d0_ref, d1_ref, x_ref, w_ref, dinv_ref, h_ref):
    dinv = _nr_rsqrt(d0_ref[...] + d1_ref[...] + 1.0)
    dinv_ref[...] = dinv
    h_ref[...] = dinv * jnp.dot(
        x_ref[...], w_ref[...], preferred_element_type=jnp.float32
    )

  return pl.pallas_call(
      body,
      grid=(_GRID,),
      in_specs=[
          pl.BlockSpec((_BN, 1), lambda i: (i, 0)),
          pl.BlockSpec((_BN, 1), lambda i: (i, 0)),
          pl.BlockSpec((_BN, _H), lambda i: (i, 0)),
          pl.BlockSpec((_H, _H), lambda i: (0, 0)),
      ],
      out_specs=[
          pl.BlockSpec((_BN, 1), lambda i: (i, 0)),
          pl.BlockSpec((_BN, _H), lambda i: (i, 0)),
      ],
      out_shape=[
          jax.ShapeDtypeStruct((_N, 1), jnp.float32),
          jax.ShapeDtypeStruct((_N, _H), jnp.float32),
      ],
  )(degp0, degp1, x, W1)


def _tc_layer(p, h, dinv, b2d, Wn):
  def body(p_ref, h_ref, dinv_ref, b_ref, w_ref, o_ref):
    acc = p_ref[0] + p_ref[1] + h_ref[...]
    xl = jnp.maximum(dinv_ref[...] * acc + b_ref[...], 0.0)
    o_ref[...] = dinv_ref[...] * jnp.dot(
        xl, w_ref[...], preferred_element_type=jnp.float32
    )

  return pl.pallas_call(
      body,
      grid=(_GRID,),
      in_specs=[
          pl.BlockSpec((2, _BN, _H), lambda i: (0, i, 0)),
          pl.BlockSpec((_BN, _H), lambda i: (i, 0)),
          pl.BlockSpec((_BN, 1), lambda i: (i, 0)),
          pl.BlockSpec((1, _H), lambda i: (0, 0)),
          pl.BlockSpec((_H, _H), lambda i: (0, 0)),
      ],
      out_specs=pl.BlockSpec((_BN, _H), lambda i: (i, 0)),
      out_shape=jax.ShapeDtypeStruct((_N, _H), jnp.float32),
  )(p, h, dinv, b2d, Wn)


def _tc_final(p, h, dinv, b2d, batch3, gm, gb, mu, var, w1, b1, w2, b2):
  def body(p_ref, h_ref, dinv_ref, b_ref, batch_ref, gm_ref, gb_ref, mu_ref,
           var_ref, w1_ref, b1_ref, w2_ref, b2_ref, xb_ref, out_ref, sums,
           counts):
    i = pl.program_id(0)

    @pl.when(i == 0)
    def _():
      sums[...] = jnp.zeros_like(sums)
      counts[...] = jnp.zeros_like(counts)

    h3 = jnp.maximum(
        dinv_ref[...] * (p_ref[0] + p_ref[1] + h_ref[...]) + b_ref[...], 0.0
    )
    g = batch_ref[0, 0, :]
    onehot = (
        g[None, :] == lax.broadcasted_iota(jnp.int32, (_G, _BN), 0)
    ).astype(jnp.float32)
    sums[...] += jnp.dot(onehot, h3, preferred_element_type=jnp.float32, precision=lax.Precision.HIGHEST)
    counts[...] += jnp.sum(onehot, axis=1, keepdims=True)

    @pl.when(i == _GRID - 1)
    def _():
      pooled = sums[...] * _nr_recip(jnp.maximum(counts[...], 1.0))
      xb = (pooled - mu_ref[...]) * _nr_rsqrt(var_ref[...] + 1e-5) * gm_ref[
          ...
      ] + gb_ref[...]
      xb_ref[...] = xb
      t1 = jnp.maximum(
          jnp.dot(xb, w1_ref[...], preferred_element_type=jnp.float32)
          + b1_ref[...],
          0.0,
      )
      out_ref[...] = jnp.maximum(
          jnp.dot(t1, w2_ref[...], preferred_element_type=jnp.float32)
          + b2_ref[...],
          0.0,
      )

  return pl.pallas_call(
      body,
      grid=(_GRID,),
      in_specs=[
          pl.BlockSpec((2, _BN, _H), lambda i: (0, i, 0)),
          pl.BlockSpec((_BN, _H), lambda i: (i, 0)),
          pl.BlockSpec((_BN, 1), lambda i: (i, 0)),
          pl.BlockSpec((1, _H), lambda i: (0, 0)),
          pl.BlockSpec((1, 1, _BN), lambda i: (i, 0, 0)),
          pl.BlockSpec((1, _H), lambda i: (0, 0)),
          pl.BlockSpec((1, _H), lambda i: (0, 0)),
          pl.BlockSpec((1, _H), lambda i: (0, 0)),
          pl.BlockSpec((1, _H), lambda i: (0, 0)),
          pl.BlockSpec((_H, _H // 2), lambda i: (0, 0)),
          pl.BlockSpec((1, _H // 2), lambda i: (0, 0)),
          pl.BlockSpec((_H // 2, 2), lambda i: (0, 0)),
          pl.BlockSpec((1, 2), lambda i: (0, 0)),
      ],
      out_specs=[
          pl.BlockSpec((_G, _H), lambda i: (0, 0)),
          pl.BlockSpec((_G, 2), lambda i: (0, 0)),
      ],
      out_shape=[
          jax.ShapeDtypeStruct((_G, _H), jnp.float32),
          jax.ShapeDtypeStruct((_G, 2), jnp.float32),
      ],
      scratch_shapes=[
          pltpu.VMEM((_G, _H), jnp.float32),
          pltpu.VMEM((_G, 1), jnp.float32),
      ],
  )(p, h, dinv, b2d, batch3, gm, gb, mu, var, w1, b1, w2, b2)


def kernel(x, edge_index, _, batch, W1, b1, W2, b2, W3, b3, bn_gamma, bn_beta,
           bn_mean, bn_var, lin1_W, lin1_b, lin2_W, lin2_b):
  src = edge_index[0]
  dst = edge_index[1]
  pad = _EPAD - _E
  ar = jnp.arange(pad, dtype=jnp.int32)
  pad_src = (ar * 613) % _N
  pad_dst = _N + (ar % _JUNK)
  src2d = jnp.concatenate([src, pad_src]).reshape(_ROWS, _LANES)
  dst2d = jnp.concatenate([dst, pad_dst]).reshape(_ROWS, _LANES)
  zeros1d = jnp.zeros((_DPT,), jnp.float32)
  zeros2d = jnp.zeros((_ZROWS, _H), jnp.float32)

  degp = _deg_partials(dst2d, zeros1d)
  dinv, h1 = _tc_first(
      degp[:_N, None], degp[_NDEG : _NDEG + _N, None], x, W1
  )
  p1 = _msg_partials(h1, src2d, dst2d, zeros2d)
  h2 = _tc_layer(p1, h1, dinv, b1[None, :], W2)
  p2 = _msg_partials(h2, src2d, dst2d, zeros2d)
  h3 = _tc_layer(p2, h2, dinv, b2[None, :], W3)
  p3 = _msg_partials(h3, src2d, dst2d, zeros2d)
  xb, out = _tc_final(
      p3, h3, dinv, b3[None, :],
      batch.reshape(_GRID, 1, _BN),
      bn_gamma[None, :], bn_beta[None, :], bn_mean[None, :], bn_var[None, :],
      lin1_W, lin1_b[None, :], lin2_W, lin2_b[None, :],
  )
  return (xb, out)
